# baseline (speedup 1.0000x reference)
import functools
import math

import jax
import jax.numpy as jnp
from jax import lax
from jax.experimental import pallas as pl
from jax.experimental.pallas import tpu as pltpu
from jax.experimental.pallas import tpu_sc as plsc

V = 1000000
E = 64
B = 4096
W = 10
NS = 5
K = 2 * W * (1 + NS)
KP = 128
EP = 2 * E

NC = 2
NSUB = 16
NW = NC * NSUB
BPW = B // NW
GB = 2
NG = BPW // GB
L = 16


def _sc_scores():
    mesh = plsc.VectorSubcoreMesh(
        core_axis_name="c", subcore_axis_name="s",
        num_cores=NC, num_subcores=NSUB)

    @functools.partial(
        pl.kernel,
        out_type=jax.ShapeDtypeStruct((B * KP,), jnp.float32),
        mesh=mesh,
        compiler_params=pltpu.CompilerParams(
            needs_layout_passes=False, use_tc_tiling_on_sc=False),
        scratch_types=[
            pltpu.VMEM((BPW,), jnp.int32),
            pltpu.VMEM((BPW,), jnp.int32),
            pltpu.VMEM((BPW, K), jnp.int32),
            pltpu.VMEM((BPW * K + L,), jnp.int32),
            pltpu.VMEM((BPW, E), jnp.bfloat16),
            pltpu.VMEM((2, GB * KP, EP), jnp.bfloat16),
            pltpu.VMEM((BPW * KP,), jnp.float32),
            pltpu.SemaphoreType.DMA,
            pltpu.SemaphoreType.DMA,
            pltpu.SemaphoreType.DMA,
        ],
    )
    def body(iw2_hbm, iwp_hbm, idx2_hbm, idxp_hbm, itab_hbm, otab_hbm,
             out_hbm, iw2_v, iwp_v, idx2_v, idxp_v, ivec_v, rows_v, sc_v,
             sem0, sem1, semi):
        wid = lax.axis_index("s") * NC + lax.axis_index("c")
        base = wid * BPW

        pltpu.sync_copy(iw2_hbm.at[pl.ds(base, BPW)], iw2_v)
        pltpu.sync_copy(iwp_hbm.at[pl.ds(base, BPW)], iwp_v)
        pltpu.sync_copy(idx2_hbm.at[pl.ds(base, BPW)], idx2_v)
        pltpu.sync_copy(idxp_hbm.at[pl.ds(base * K, BPW * K)],
                        idxp_v.at[pl.ds(0, BPW * K)])

        pltpu.async_copy(
            itab_hbm.at[iw2_v], rows_v.at[0, pl.ds(0, BPW)], semi).wait()

        @pl.loop(0, BPW // L)
        def _(kk):
            ipv = iwp_v[pl.ds(kk * L, L)]
            for i in range(L):
                b = kk * L + i
                ioff = (ipv[i] & 1) * E
                for e in range(E // (2 * L)):
                    ivec_v[b, pl.ds(2 * L * e, 2 * L)] = \
                        rows_v[0, b, pl.ds(ioff + 2 * L * e, 2 * L)]

        def fire(g, slot, sem):
            for j in range(GB):
                pltpu.async_copy(
                    otab_hbm.at[idx2_v.at[g * GB + j]],
                    rows_v.at[slot, pl.ds(j * KP, K)], sem)

        def drain(slot, sem):
            pltpu.make_async_copy(
                otab_hbm.at[pl.ds(0, GB * K)],
                rows_v.at[slot, pl.ds(0, GB * K)], sem).wait()

        def compute(g, slot):
            for j in range(GB):
                b = g * GB + j
                iv = []
                for e in range(E // (2 * L)):
                    iv.extend(plsc.unpack(
                        ivec_v[b, pl.ds(2 * L * e, 2 * L)],
                        format=plsc.PackFormat.INTERLEAVED))

                @pl.loop(0, KP // L)
                def _(k):
                    pv = idxp_v[pl.ds(b * K + k * L, L)]
                    svec = jnp.zeros((L,), jnp.float32)
                    lane = lax.iota(jnp.int32, L)
                    for i in range(L):
                        roff = (pv[i] & 1) * E
                        rr = j * KP + k * L + i
                        acc = jnp.zeros((L,), jnp.float32)
                        for e in range(E // (2 * L)):
                            ra, rb = plsc.unpack(
                                rows_v[slot, rr, pl.ds(roff + 2 * L * e,
                                                       2 * L)],
                                format=plsc.PackFormat.INTERLEAVED)
                            acc = acc + ra * iv[2 * e] + rb * iv[2 * e + 1]
                        svec = jnp.where(lane == i, jnp.sum(acc), svec)
                    sc_v[pl.ds(b * KP + k * L, L)] = svec

        fire(0, 0, sem0)

        @pl.loop(0, NG - 2, step=2)
        def _(g):
            fire(g + 1, 1, sem1)
            drain(0, sem0)
            compute(g, 0)
            fire(g + 2, 0, sem0)
            drain(1, sem1)
            compute(g + 1, 1)

        fire(NG - 1, 1, sem1)
        drain(0, sem0)
        compute(NG - 2, 0)
        drain(1, sem1)
        compute(NG - 1, 1)

        pltpu.sync_copy(sc_v, out_hbm.at[pl.ds(base * KP, BPW * KP)])

    return body


_HC = 2048


def _tc_relayout(tT):
    grid = (V + 2 * _HC - 1) // (2 * _HC)

    def body(t_ref, o_ref):
        t = t_ref[...]
        pairs = jnp.concatenate([t[:, :_HC].T, t[:, _HC:].T], axis=1)
        o_ref[...] = pairs.astype(jnp.bfloat16)

    return pl.pallas_call(
        body,
        grid=(grid,),
        in_specs=[pl.BlockSpec((E, 2 * _HC), lambda j: (0, j))],
        out_specs=pl.BlockSpec((_HC, 128), lambda j: (j, 0)),
        out_shape=jax.ShapeDtypeStruct((grid * _HC, EP), jnp.bfloat16),
    )(tT)


def _tc_loss(scores2d):
    def body(s_ref, o_ref):
        x = s_ref[...]
        col = lax.broadcasted_iota(jnp.int32, (B, KP), 1)
        x = jnp.where(col < 2 * W, x, -x)
        x = jnp.where(col < K, x, 0.0)
        ls = jnp.minimum(x, 0.0) - jnp.log1p(jnp.exp(-jnp.abs(x)))
        total = jnp.sum(ls) + B * (KP - K) * math.log(2.0)
        o_ref[0, 0] = -total * (1.0 / (B * 2 * W))

    out = pl.pallas_call(
        body,
        out_shape=jax.ShapeDtypeStruct((1, 1), jnp.float32),
        out_specs=pl.BlockSpec(memory_space=pltpu.SMEM),
    )(scores2d)
    return out.reshape(())


def kernel(i_word, o_word, n_word, i_table, o_table):
    idx = jnp.concatenate([o_word, n_word], axis=1).astype(jnp.int32)
    iw = i_word.astype(jnp.int32)
    i2 = _tc_relayout(i_table.T)
    o2 = _tc_relayout(o_table.T)
    iw2 = ((iw >> 12) << 11) | (iw & (_HC - 1))
    iwp = (iw >> 11) & 1
    idx2 = ((idx >> 12) << 11) | (idx & (_HC - 1))
    idxp = (idx >> 11) & 1
    scores = _sc_scores()(iw2, iwp, idx2, idxp.reshape(-1), i2, o2)
    return _tc_loss(scores.reshape(B, KP))


# speedup vs baseline: 1.7568x; 1.7568x over previous
import functools
import math

import jax
import jax.numpy as jnp
from jax import lax
from jax.experimental import pallas as pl
from jax.experimental.pallas import tpu as pltpu
from jax.experimental.pallas import tpu_sc as plsc

V = 1000000
E = 64
B = 4096
W = 10
NS = 5
K = 2 * W * (1 + NS)
KP = 128
EP = 2 * E

NC = 2
NSUB = 16
NW = NC * NSUB
BPW = B // NW
GB = 2
NG = BPW // GB
L = 16


def _sc_scores():
    mesh = plsc.VectorSubcoreMesh(
        core_axis_name="c", subcore_axis_name="s",
        num_cores=NC, num_subcores=NSUB)

    @functools.partial(
        pl.kernel,
        out_type=jax.ShapeDtypeStruct((B * KP,), jnp.float32),
        mesh=mesh,
        compiler_params=pltpu.CompilerParams(
            needs_layout_passes=False, use_tc_tiling_on_sc=False),
        scratch_types=[
            pltpu.VMEM((BPW,), jnp.int32),
            pltpu.VMEM((BPW,), jnp.int32),
            pltpu.VMEM((BPW, K), jnp.int32),
            pltpu.VMEM((BPW * K + L,), jnp.int32),
            pltpu.VMEM((BPW, E), jnp.float32),
            pltpu.VMEM((2, GB * KP, EP), jnp.float32),
            pltpu.VMEM((BPW * KP,), jnp.float32),
            pltpu.SemaphoreType.DMA,
            pltpu.SemaphoreType.DMA,
            pltpu.SemaphoreType.DMA,
        ],
    )
    def body(iw2_hbm, iwp_hbm, idx2_hbm, idxp_hbm, itab_hbm, otab_hbm,
             out_hbm, iw2_v, iwp_v, idx2_v, idxp_v, ivec_v, rows_v, sc_v,
             sem0, sem1, semi):
        wid = lax.axis_index("s") * NC + lax.axis_index("c")
        base = wid * BPW

        pltpu.sync_copy(iw2_hbm.at[pl.ds(base, BPW)], iw2_v)
        pltpu.sync_copy(iwp_hbm.at[pl.ds(base, BPW)], iwp_v)
        pltpu.sync_copy(idx2_hbm.at[pl.ds(base, BPW)], idx2_v)
        pltpu.sync_copy(idxp_hbm.at[pl.ds(base * K, BPW * K)],
                        idxp_v.at[pl.ds(0, BPW * K)])

        pltpu.async_copy(
            itab_hbm.at[iw2_v], rows_v.at[0, pl.ds(0, BPW)], semi).wait()

        @pl.loop(0, BPW // L)
        def _(kk):
            ipv = iwp_v[pl.ds(kk * L, L)]
            for i in range(L):
                b = kk * L + i
                ioff = (ipv[i] & 1) * E
                for e in range(E // L):
                    ivec_v[b, pl.ds(L * e, L)] = \
                        rows_v[0, b, pl.ds(ioff + L * e, L)]

        def fire(g, slot, sem):
            for j in range(GB):
                pltpu.async_copy(
                    otab_hbm.at[idx2_v.at[g * GB + j]],
                    rows_v.at[slot, pl.ds(j * KP, K)], sem)

        def drain(slot, sem):
            pltpu.make_async_copy(
                otab_hbm.at[pl.ds(0, GB * K)],
                rows_v.at[slot, pl.ds(0, GB * K)], sem).wait()

        def compute(g, slot):
            for j in range(GB):
                b = g * GB + j
                iv = [ivec_v[b, pl.ds(L * e, L)] for e in range(E // L)]

                @pl.loop(0, KP // L)
                def _(k):
                    pv = idxp_v[pl.ds(b * K + k * L, L)]
                    svec = jnp.zeros((L,), jnp.float32)
                    lane = lax.iota(jnp.int32, L)
                    for i in range(L):
                        roff = (pv[i] & 1) * E
                        rr = j * KP + k * L + i
                        row = [rows_v[slot, rr, pl.ds(roff + L * e, L)]
                               for e in range(E // L)]
                        acc = row[0] * iv[0] + row[1] * iv[1]
                        acc = acc + row[2] * iv[2] + row[3] * iv[3]
                        svec = jnp.where(lane == i, jnp.sum(acc), svec)
                    sc_v[pl.ds(b * KP + k * L, L)] = svec

        fire(0, 0, sem0)

        @pl.loop(0, NG - 2, step=2)
        def _(g):
            fire(g + 1, 1, sem1)
            drain(0, sem0)
            compute(g, 0)
            fire(g + 2, 0, sem0)
            drain(1, sem1)
            compute(g + 1, 1)

        fire(NG - 1, 1, sem1)
        drain(0, sem0)
        compute(NG - 2, 0)
        drain(1, sem1)
        compute(NG - 1, 1)

        pltpu.sync_copy(sc_v, out_hbm.at[pl.ds(base * KP, BPW * KP)])

    return body


_HC = 2048


def _tc_relayout(tT):
    grid = (V + 2 * _HC - 1) // (2 * _HC)

    def body(t_ref, o_ref):
        t = t_ref[...]
        o_ref[...] = jnp.concatenate([t[:, :_HC].T, t[:, _HC:].T], axis=1)

    return pl.pallas_call(
        body,
        grid=(grid,),
        in_specs=[pl.BlockSpec((E, 2 * _HC), lambda j: (0, j))],
        out_specs=pl.BlockSpec((_HC, 128), lambda j: (j, 0)),
        out_shape=jax.ShapeDtypeStruct((grid * _HC, EP), jnp.float32),
    )(tT)


def _tc_loss(scores2d):
    def body(s_ref, o_ref):
        x = s_ref[...]
        col = lax.broadcasted_iota(jnp.int32, (B, KP), 1)
        x = jnp.where(col < 2 * W, x, -x)
        x = jnp.where(col < K, x, 0.0)
        ls = jnp.minimum(x, 0.0) - jnp.log1p(jnp.exp(-jnp.abs(x)))
        total = jnp.sum(ls) + B * (KP - K) * math.log(2.0)
        o_ref[0, 0] = -total * (1.0 / (B * 2 * W))

    out = pl.pallas_call(
        body,
        out_shape=jax.ShapeDtypeStruct((1, 1), jnp.float32),
        out_specs=pl.BlockSpec(memory_space=pltpu.SMEM),
    )(scores2d)
    return out.reshape(())


def kernel(i_word, o_word, n_word, i_table, o_table):
    idx = jnp.concatenate([o_word, n_word], axis=1).astype(jnp.int32)
    iw = i_word.astype(jnp.int32)
    i2 = _tc_relayout(i_table.T)
    o2 = _tc_relayout(o_table.T)
    iw2 = ((iw >> 12) << 11) | (iw & (_HC - 1))
    iwp = (iw >> 11) & 1
    idx2 = ((idx >> 12) << 11) | (idx & (_HC - 1))
    idxp = (idx >> 11) & 1
    scores = _sc_scores()(iw2, iwp, idx2, idxp.reshape(-1), i2, o2)
    return _tc_loss(scores.reshape(B, KP))


# speedup vs baseline: 2.1638x; 1.2317x over previous
import functools
import math

import jax
import jax.numpy as jnp
from jax import lax
from jax.experimental import pallas as pl
from jax.experimental.pallas import tpu as pltpu
from jax.experimental.pallas import tpu_sc as plsc

V = 1000000
E = 64
B = 4096
W = 10
NS = 5
K = 2 * W * (1 + NS)
KP = 128
EP = 2 * E

NC = 2
NSUB = 16
NW = NC * NSUB
BPW = B // NW
GB = 2
NG = BPW // GB
L = 16


def _sc_scores():
    mesh = plsc.VectorSubcoreMesh(
        core_axis_name="c", subcore_axis_name="s",
        num_cores=NC, num_subcores=NSUB)

    @functools.partial(
        pl.kernel,
        out_type=jax.ShapeDtypeStruct((B * KP,), jnp.float32),
        mesh=mesh,
        compiler_params=pltpu.CompilerParams(
            needs_layout_passes=False, use_tc_tiling_on_sc=False),
        scratch_types=[
            pltpu.VMEM((BPW,), jnp.int32),
            pltpu.VMEM((BPW,), jnp.int32),
            pltpu.VMEM((BPW, K), jnp.int32),
            pltpu.VMEM((BPW * K + L,), jnp.int32),
            pltpu.VMEM((BPW, E), jnp.float32),
            pltpu.VMEM((2, GB * KP, EP), jnp.float32),
            pltpu.VMEM((BPW * KP,), jnp.float32),
            pltpu.SemaphoreType.DMA,
            pltpu.SemaphoreType.DMA,
            pltpu.SemaphoreType.DMA,
        ],
    )
    def body(iw2_hbm, iwp_hbm, idx2_hbm, idxp_hbm, itab_hbm, otab_hbm,
             out_hbm, iw2_v, iwp_v, idx2_v, idxp_v, ivec_v, rows_v, sc_v,
             sem0, sem1, semi):
        wid = lax.axis_index("s") * NC + lax.axis_index("c")
        base = wid * BPW

        pltpu.sync_copy(iw2_hbm.at[pl.ds(base, BPW)], iw2_v)
        pltpu.sync_copy(iwp_hbm.at[pl.ds(base, BPW)], iwp_v)
        pltpu.sync_copy(idx2_hbm.at[pl.ds(base, BPW)], idx2_v)
        pltpu.sync_copy(idxp_hbm.at[pl.ds(base * K, BPW * K)],
                        idxp_v.at[pl.ds(0, BPW * K)])

        pltpu.async_copy(
            itab_hbm.at[iw2_v], rows_v.at[0, pl.ds(0, BPW)], semi).wait()

        @pl.loop(0, BPW // L)
        def _(kk):
            ipv = iwp_v[pl.ds(kk * L, L)]
            for i in range(L):
                b = kk * L + i
                ioff = (ipv[i] & 1) * E
                for e in range(E // L):
                    ivec_v[b, pl.ds(L * e, L)] = \
                        rows_v[0, b, pl.ds(ioff + L * e, L)]

        def fire(g, slot, sem):
            for j in range(GB):
                pltpu.async_copy(
                    otab_hbm.at[idx2_v.at[g * GB + j]],
                    rows_v.at[slot, pl.ds(j * KP, K)], sem)

        def drain(slot, sem):
            pltpu.make_async_copy(
                otab_hbm.at[pl.ds(0, GB * K)],
                rows_v.at[slot, pl.ds(0, GB * K)], sem).wait()

        def compute(g, slot):
            for j in range(GB):
                b = g * GB + j
                iv = [ivec_v[b, pl.ds(L * e, L)] for e in range(E // L)]

                @pl.loop(0, KP // L)
                def _(k):
                    pv = idxp_v[pl.ds(b * K + k * L, L)]
                    svec = jnp.zeros((L,), jnp.float32)
                    lane = lax.iota(jnp.int32, L)
                    for i in range(L):
                        roff = (pv[i] & 1) * E
                        rr = j * KP + k * L + i
                        row = [rows_v[slot, rr, pl.ds(roff + L * e, L)]
                               for e in range(E // L)]
                        acc = row[0] * iv[0] + row[1] * iv[1]
                        acc = acc + row[2] * iv[2] + row[3] * iv[3]
                        svec = jnp.where(lane == i, jnp.sum(acc), svec)
                    sc_v[pl.ds(b * KP + k * L, L)] = svec

        fire(0, 0, sem0)

        @pl.loop(0, NG - 2, step=2)
        def _(g):
            fire(g + 1, 1, sem1)
            drain(0, sem0)
            compute(g, 0)
            fire(g + 2, 0, sem0)
            drain(1, sem1)
            compute(g + 1, 1)

        fire(NG - 1, 1, sem1)
        drain(0, sem0)
        compute(NG - 2, 0)
        drain(1, sem1)
        compute(NG - 1, 1)

        pltpu.sync_copy(sc_v, out_hbm.at[pl.ds(base * KP, BPW * KP)])

    return body


_HC = 2048
_HSH = 11


def _tc_relayout(iT, oT):
    grid = (V + 2 * _HC - 1) // (2 * _HC)

    def body(ti_ref, to_ref, oi_ref, oo_ref):
        ti = ti_ref[...]
        oi_ref[...] = jnp.concatenate([ti[:, :_HC].T, ti[:, _HC:].T], axis=1)
        to = to_ref[...]
        oo_ref[...] = jnp.concatenate([to[:, :_HC].T, to[:, _HC:].T], axis=1)

    spec_in = pl.BlockSpec((E, 2 * _HC), lambda j: (0, j))
    spec_out = pl.BlockSpec((_HC, 128), lambda j: (j, 0))
    oshape = jax.ShapeDtypeStruct((grid * _HC, EP), jnp.float32)
    return pl.pallas_call(
        body,
        grid=(grid,),
        in_specs=[spec_in, spec_in],
        out_specs=[spec_out, spec_out],
        out_shape=[oshape, oshape],
    )(iT, oT)


def _tc_loss(scores2d):
    def body(s_ref, o_ref):
        x = s_ref[...]
        col = lax.broadcasted_iota(jnp.int32, (B, KP), 1)
        x = jnp.where(col < 2 * W, x, -x)
        x = jnp.where(col < K, x, 0.0)
        ls = jnp.minimum(x, 0.0) - jnp.log1p(jnp.exp(-jnp.abs(x)))
        total = jnp.sum(ls) + B * (KP - K) * math.log(2.0)
        o_ref[0, 0] = -total * (1.0 / (B * 2 * W))

    out = pl.pallas_call(
        body,
        out_shape=jax.ShapeDtypeStruct((1, 1), jnp.float32),
        out_specs=pl.BlockSpec(memory_space=pltpu.SMEM),
    )(scores2d)
    return out.reshape(())


def kernel(i_word, o_word, n_word, i_table, o_table):
    idx = jnp.concatenate([o_word, n_word], axis=1).astype(jnp.int32)
    iw = i_word.astype(jnp.int32)
    i2, o2 = _tc_relayout(i_table.T, o_table.T)
    iw2 = ((iw >> (_HSH + 1)) << _HSH) | (iw & (_HC - 1))
    iwp = (iw >> _HSH) & 1
    idx2 = ((idx >> (_HSH + 1)) << _HSH) | (idx & (_HC - 1))
    idxp = (idx >> _HSH) & 1
    scores = _sc_scores()(iw2, iwp, idx2, idxp.reshape(-1), i2, o2)
    return _tc_loss(scores.reshape(B, KP))


# speedup vs baseline: 2.4710x; 1.1420x over previous
import functools
import math

import jax
import jax.numpy as jnp
from jax import lax
from jax.experimental import pallas as pl
from jax.experimental.pallas import tpu as pltpu
from jax.experimental.pallas import tpu_sc as plsc

V = 1000000
E = 64
B = 4096
W = 10
NS = 5
K = 2 * W * (1 + NS)
KP = 128
EP = 2 * E

NC = 2
NSUB = 16
NW = NC * NSUB
BPW = B // NW
GB = 2
NG = BPW // GB
L = 16


def _sc_scores():
    mesh = plsc.VectorSubcoreMesh(
        core_axis_name="c", subcore_axis_name="s",
        num_cores=NC, num_subcores=NSUB)

    @functools.partial(
        pl.kernel,
        out_type=jax.ShapeDtypeStruct((B * KP,), jnp.float32),
        mesh=mesh,
        compiler_params=pltpu.CompilerParams(
            needs_layout_passes=False, use_tc_tiling_on_sc=False),
        scratch_types=[
            pltpu.VMEM((BPW,), jnp.int32),
            pltpu.VMEM((BPW,), jnp.int32),
            pltpu.VMEM((BPW, K), jnp.int32),
            pltpu.VMEM((BPW * K + L,), jnp.int32),
            pltpu.VMEM((BPW, E), jnp.float32),
            pltpu.VMEM((2, GB * KP, EP), jnp.float32),
            pltpu.VMEM((BPW * KP,), jnp.float32),
            pltpu.SemaphoreType.DMA,
            pltpu.SemaphoreType.DMA,
            pltpu.SemaphoreType.DMA,
        ],
    )
    def body(iw2_hbm, iwp_hbm, idx2_hbm, idxp_hbm, itab_hbm, otab_hbm,
             out_hbm, iw2_v, iwp_v, idx2_v, idxp_v, ivec_v, rows_v, sc_v,
             sem0, sem1, semi):
        wid = lax.axis_index("s") * NC + lax.axis_index("c")
        base = wid * BPW

        pltpu.sync_copy(iw2_hbm.at[pl.ds(base, BPW)], iw2_v)
        pltpu.sync_copy(iwp_hbm.at[pl.ds(base, BPW)], iwp_v)
        pltpu.sync_copy(idx2_hbm.at[pl.ds(base, BPW)], idx2_v)
        pltpu.sync_copy(idxp_hbm.at[pl.ds(base * K, BPW * K)],
                        idxp_v.at[pl.ds(0, BPW * K)])

        pltpu.async_copy(
            itab_hbm.at[iw2_v], rows_v.at[0, pl.ds(0, BPW)], semi).wait()

        @pl.loop(0, BPW // L)
        def _(kk):
            ipv = iwp_v[pl.ds(kk * L, L)]
            for i in range(L):
                b = kk * L + i
                ioff = (ipv[i] & 1) * E
                for e in range(E // L):
                    ivec_v[b, pl.ds(L * e, L)] = \
                        rows_v[0, b, pl.ds(ioff + L * e, L)]

        def fire(g, slot, sem):
            for j in range(GB):
                pltpu.async_copy(
                    otab_hbm.at[idx2_v.at[g * GB + j]],
                    rows_v.at[slot, pl.ds(j * KP, K)], sem)

        def drain(slot, sem):
            pltpu.make_async_copy(
                otab_hbm.at[pl.ds(0, GB * K)],
                rows_v.at[slot, pl.ds(0, GB * K)], sem).wait()

        def compute(g, slot):
            for j in range(GB):
                b = g * GB + j
                iv = [ivec_v[b, pl.ds(L * e, L)] for e in range(E // L)]

                @pl.loop(0, KP // L)
                def _(k):
                    pv = idxp_v[pl.ds(b * K + k * L, L)]
                    svec = jnp.zeros((L,), jnp.float32)
                    lane = lax.iota(jnp.int32, L)
                    for i in range(L):
                        roff = (pv[i] & 1) * E
                        rr = j * KP + k * L + i
                        row = [rows_v[slot, rr, pl.ds(roff + L * e, L)]
                               for e in range(E // L)]
                        acc = row[0] * iv[0] + row[1] * iv[1]
                        acc = acc + row[2] * iv[2] + row[3] * iv[3]
                        svec = jnp.where(lane == i, jnp.sum(acc), svec)
                    sc_v[pl.ds(b * KP + k * L, L)] = svec

        fire(0, 0, sem0)

        @pl.loop(0, NG - 2, step=2)
        def _(g):
            fire(g + 1, 1, sem1)
            drain(0, sem0)
            compute(g, 0)
            fire(g + 2, 0, sem0)
            drain(1, sem1)
            compute(g + 1, 1)

        fire(NG - 1, 1, sem1)
        drain(0, sem0)
        compute(NG - 2, 0)
        drain(1, sem1)
        compute(NG - 1, 1)

        pltpu.sync_copy(sc_v, out_hbm.at[pl.ds(base * KP, BPW * KP)])

    return body


_HC = 4096
_HSH = 12


def _tc_relayout(iT, oT):
    grid = (V + 2 * _HC - 1) // (2 * _HC)

    def body(ti_ref, to_ref, oi_ref, oo_ref):
        ti = ti_ref[...]
        oi_ref[...] = jnp.concatenate([ti[:, :_HC].T, ti[:, _HC:].T], axis=1)
        to = to_ref[...]
        oo_ref[...] = jnp.concatenate([to[:, :_HC].T, to[:, _HC:].T], axis=1)

    spec_in = pl.BlockSpec((E, 2 * _HC), lambda j: (0, j))
    spec_out = pl.BlockSpec((_HC, 128), lambda j: (j, 0))
    oshape = jax.ShapeDtypeStruct((grid * _HC, EP), jnp.float32)
    return pl.pallas_call(
        body,
        grid=(grid,),
        in_specs=[spec_in, spec_in],
        out_specs=[spec_out, spec_out],
        out_shape=[oshape, oshape],
    )(iT, oT)


def _tc_loss(scores2d):
    def body(s_ref, o_ref):
        x = s_ref[...]
        col = lax.broadcasted_iota(jnp.int32, (B, KP), 1)
        x = jnp.where(col < 2 * W, x, -x)
        x = jnp.where(col < K, x, 0.0)
        ls = jnp.minimum(x, 0.0) - jnp.log1p(jnp.exp(-jnp.abs(x)))
        total = jnp.sum(ls) + B * (KP - K) * math.log(2.0)
        o_ref[0, 0] = -total * (1.0 / (B * 2 * W))

    out = pl.pallas_call(
        body,
        out_shape=jax.ShapeDtypeStruct((1, 1), jnp.float32),
        out_specs=pl.BlockSpec(memory_space=pltpu.SMEM),
    )(scores2d)
    return out.reshape(())


def kernel(i_word, o_word, n_word, i_table, o_table):
    idx = jnp.concatenate([o_word, n_word], axis=1).astype(jnp.int32)
    iw = i_word.astype(jnp.int32)
    i2, o2 = _tc_relayout(i_table.T, o_table.T)
    iw2 = ((iw >> (_HSH + 1)) << _HSH) | (iw & (_HC - 1))
    iwp = (iw >> _HSH) & 1
    idx2 = ((idx >> (_HSH + 1)) << _HSH) | (idx & (_HC - 1))
    idxp = (idx >> _HSH) & 1
    scores = _sc_scores()(iw2, iwp, idx2, idxp.reshape(-1), i2, o2)
    return _tc_loss(scores.reshape(B, KP))


# speedup vs baseline: 2.6851x; 1.0866x over previous
import functools
import math

import jax
import jax.numpy as jnp
from jax import lax
from jax.experimental import pallas as pl
from jax.experimental.pallas import tpu as pltpu
from jax.experimental.pallas import tpu_sc as plsc

V = 1000000
E = 64
B = 4096
W = 10
NS = 5
K = 2 * W * (1 + NS)
KP = 128
EP = 2 * E

NC = 2
NSUB = 16
NW = NC * NSUB
BPW = B // NW
GB = 2
NG = BPW // GB
L = 16


def _sc_scores():
    mesh = plsc.VectorSubcoreMesh(
        core_axis_name="c", subcore_axis_name="s",
        num_cores=NC, num_subcores=NSUB)

    @functools.partial(
        pl.kernel,
        out_type=jax.ShapeDtypeStruct((B * KP,), jnp.float32),
        mesh=mesh,
        compiler_params=pltpu.CompilerParams(
            needs_layout_passes=False, use_tc_tiling_on_sc=False),
        scratch_types=[
            pltpu.VMEM((BPW,), jnp.int32),
            pltpu.VMEM((BPW,), jnp.int32),
            pltpu.VMEM((BPW, K), jnp.int32),
            pltpu.VMEM((BPW * K + L,), jnp.int32),
            pltpu.VMEM((BPW, E), jnp.float32),
            pltpu.VMEM((2, GB * KP, EP), jnp.float32),
            pltpu.VMEM((BPW * KP,), jnp.float32),
            pltpu.SemaphoreType.DMA,
            pltpu.SemaphoreType.DMA,
            pltpu.SemaphoreType.DMA,
        ],
    )
    def body(iw2_hbm, iwp_hbm, idx2_hbm, idxp_hbm, itab_hbm, otab_hbm,
             out_hbm, iw2_v, iwp_v, idx2_v, idxp_v, ivec_v, rows_v, sc_v,
             sem0, sem1, semi):
        wid = lax.axis_index("s") * NC + lax.axis_index("c")
        base = wid * BPW

        pltpu.sync_copy(iw2_hbm.at[pl.ds(base, BPW)], iw2_v)
        pltpu.sync_copy(iwp_hbm.at[pl.ds(base, BPW)], iwp_v)
        pltpu.sync_copy(idx2_hbm.at[pl.ds(base, BPW)], idx2_v)
        pltpu.sync_copy(idxp_hbm.at[pl.ds(base * K, BPW * K)],
                        idxp_v.at[pl.ds(0, BPW * K)])

        pltpu.async_copy(
            itab_hbm.at[iw2_v], rows_v.at[0, pl.ds(0, BPW)], semi).wait()

        @pl.loop(0, BPW // L)
        def _(kk):
            ipv = iwp_v[pl.ds(kk * L, L)]
            for i in range(L):
                b = kk * L + i
                ioff = (ipv[i] & 1) * E
                for e in range(E // L):
                    ivec_v[b, pl.ds(L * e, L)] = \
                        rows_v[0, b, pl.ds(ioff + L * e, L)]

        def fire(g, slot, sem):
            for j in range(GB):
                pltpu.async_copy(
                    otab_hbm.at[idx2_v.at[g * GB + j]],
                    rows_v.at[slot, pl.ds(j * KP, K)], sem)

        def drain(slot, sem):
            pltpu.make_async_copy(
                otab_hbm.at[pl.ds(0, GB * K)],
                rows_v.at[slot, pl.ds(0, GB * K)], sem).wait()

        def compute(g, slot):
            for j in range(GB):
                b = g * GB + j
                iv = [ivec_v[b, pl.ds(L * e, L)] for e in range(E // L)]

                @pl.loop(0, KP // L)
                def _(k):
                    pv = idxp_v[pl.ds(b * K + k * L, L)]
                    svec = jnp.zeros((L,), jnp.float32)
                    lane = lax.iota(jnp.int32, L)
                    for i in range(L):
                        roff = (pv[i] & 1) * E
                        rr = j * KP + k * L + i
                        row = [rows_v[slot, rr, pl.ds(roff + L * e, L)]
                               for e in range(E // L)]
                        acc = row[0] * iv[0] + row[1] * iv[1]
                        acc = acc + row[2] * iv[2] + row[3] * iv[3]
                        svec = jnp.where(lane == i, jnp.sum(acc), svec)
                    sc_v[pl.ds(b * KP + k * L, L)] = svec

        fire(0, 0, sem0)

        @pl.loop(0, NG - 2, step=2)
        def _(g):
            fire(g + 1, 1, sem1)
            drain(0, sem0)
            compute(g, 0)
            fire(g + 2, 0, sem0)
            drain(1, sem1)
            compute(g + 1, 1)

        fire(NG - 1, 1, sem1)
        drain(0, sem0)
        compute(NG - 2, 0)
        drain(1, sem1)
        compute(NG - 1, 1)

        pltpu.sync_copy(sc_v, out_hbm.at[pl.ds(base * KP, BPW * KP)])

    return body


_HC = 8192
_HSH = 13


def _tc_relayout(iT, oT):
    grid = (V + 2 * _HC - 1) // (2 * _HC)

    def body(ti_ref, to_ref, oi_ref, oo_ref):
        ti = ti_ref[...]
        oi_ref[...] = jnp.concatenate([ti[:, :_HC].T, ti[:, _HC:].T], axis=1)
        to = to_ref[...]
        oo_ref[...] = jnp.concatenate([to[:, :_HC].T, to[:, _HC:].T], axis=1)

    spec_in = pl.BlockSpec((E, 2 * _HC), lambda j: (0, j))
    spec_out = pl.BlockSpec((_HC, 128), lambda j: (j, 0))
    oshape = jax.ShapeDtypeStruct((grid * _HC, EP), jnp.float32)
    return pl.pallas_call(
        body,
        grid=(grid,),
        in_specs=[spec_in, spec_in],
        out_specs=[spec_out, spec_out],
        out_shape=[oshape, oshape],
    )(iT, oT)


def _tc_loss(scores2d):
    def body(s_ref, o_ref):
        x = s_ref[...]
        col = lax.broadcasted_iota(jnp.int32, (B, KP), 1)
        x = jnp.where(col < 2 * W, x, -x)
        x = jnp.where(col < K, x, 0.0)
        ls = jnp.minimum(x, 0.0) - jnp.log1p(jnp.exp(-jnp.abs(x)))
        total = jnp.sum(ls) + B * (KP - K) * math.log(2.0)
        o_ref[0, 0] = -total * (1.0 / (B * 2 * W))

    out = pl.pallas_call(
        body,
        out_shape=jax.ShapeDtypeStruct((1, 1), jnp.float32),
        out_specs=pl.BlockSpec(memory_space=pltpu.SMEM),
    )(scores2d)
    return out.reshape(())


def kernel(i_word, o_word, n_word, i_table, o_table):
    idx = jnp.concatenate([o_word, n_word], axis=1).astype(jnp.int32)
    iw = i_word.astype(jnp.int32)
    i2, o2 = _tc_relayout(i_table.T, o_table.T)
    iw2 = ((iw >> (_HSH + 1)) << _HSH) | (iw & (_HC - 1))
    iwp = (iw >> _HSH) & 1
    idx2 = ((idx >> (_HSH + 1)) << _HSH) | (idx & (_HC - 1))
    idxp = (idx >> _HSH) & 1
    scores = _sc_scores()(iw2, iwp, idx2, idxp.reshape(-1), i2, o2)
    return _tc_loss(scores.reshape(B, KP))
